# baseline (speedup 1.0000x reference)
import functools

import jax
import jax.numpy as jnp
from jax import lax
from jax.experimental import pallas as pl
from jax.experimental.pallas import tpu as pltpu
from jax.experimental.pallas import tpu_sc as plsc

B = 256
H, W = 121, 240
G = H * W
NM = 12
NC, NS, L = 2, 16, 16
NW = NC * NS
CHUNK = 1024
NCHUNK = 29
NVEC = CHUNK // L
REG = 1e-9
NT = 8
TBL_WORDS = NM * CHUNK


def _sc_body(t2m_mu, t2m_sig, tp_mu, tp_sig, month,
             t2m_lw, t2m_lb, t2m_sw, t2m_sb,
             tp_lw, tp_lb, tp_sw, tp_sb,
             o_t2m_loc, o_t2m_scale, o_tp_loc, o_tp_scale,
             month_v, tbl_v,
             mu1_v, sig1_v, mu2_v, sig2_v,
             ol1_v, os1_v, ol2_v, os2_v,
             sem_in0, sem_in1, sem_out0, sem_out1):
    sem_in = (sem_in0, sem_in1)
    sem_out = (sem_out0, sem_out1)
    wid = lax.axis_index("s") * NC + lax.axis_index("c")
    off = jnp.minimum(wid * CHUNK, G - CHUNK)

    @pl.when(wid < NCHUNK)
    def _active():
        _worker(t2m_mu, t2m_sig, tp_mu, tp_sig, month,
                t2m_lw, t2m_lb, t2m_sw, t2m_sb, tp_lw, tp_lb, tp_sw, tp_sb,
                o_t2m_loc, o_t2m_scale, o_tp_loc, o_tp_scale,
                month_v, tbl_v, mu1_v, sig1_v, mu2_v, sig2_v,
                ol1_v, os1_v, ol2_v, os2_v, sem_in, sem_out, off)


def _worker(t2m_mu, t2m_sig, tp_mu, tp_sig, month,
            t2m_lw, t2m_lb, t2m_sw, t2m_sb, tp_lw, tp_lb, tp_sw, tp_sb,
            o_t2m_loc, o_t2m_scale, o_tp_loc, o_tp_scale,
            month_v, tbl_v, mu1_v, sig1_v, mu2_v, sig2_v,
            ol1_v, os1_v, ol2_v, os2_v, sem_in, sem_out, off):
    tables = (t2m_lw, t2m_lb, t2m_sw, t2m_sb, tp_lw, tp_lb, tp_sw, tp_sb)
    for t in range(NT):
        hs = []
        for r in range(NM):
            hs.append(pltpu.async_copy(
                tables[t].at[pl.ds(r * G + off, CHUNK)],
                tbl_v.at[pl.ds(t * TBL_WORDS + r * CHUNK, CHUNK)],
                sem_in[0]))
        for h in hs:
            h.wait()
    pltpu.sync_copy(month, month_v.at[pl.ds(0, B)])

    ins = (t2m_mu, t2m_sig, tp_mu, tp_sig)
    in_bufs = (mu1_v, sig1_v, mu2_v, sig2_v)
    outs = (o_t2m_loc, o_t2m_scale, o_tp_loc, o_tp_scale)
    out_bufs = (ol1_v, os1_v, ol2_v, os2_v)

    def load_sample(b, slot):
        hs = []
        for src, buf in zip(ins, in_bufs):
            hs.append(pltpu.async_copy(
                src.at[pl.ds(b * G + off, CHUNK)],
                buf.at[pl.ds(slot * CHUNK, CHUNK)], sem_in[slot]))
        return hs

    def wait_in(slot):
        for src, buf in zip(ins, in_bufs):
            pltpu.make_async_copy(
                src.at[pl.ds(0, CHUNK)],
                buf.at[pl.ds(slot * CHUNK, CHUNK)], sem_in[slot]).wait()

    def wait_out(slot):
        for dst, buf in zip(outs, out_bufs):
            pltpu.make_async_copy(
                buf.at[pl.ds(slot * CHUNK, CHUNK)],
                dst.at[pl.ds(0, CHUNK)], sem_out[slot]).wait()

    def compute(b, slot):
        m = month_v[pl.ds(b, L)][0]
        mb = m * CHUNK
        for j in range(NVEC):
            jo = j * L
            sl = pl.ds(slot * CHUNK + jo, L)
            lw = tbl_v[pl.ds(0 * TBL_WORDS + mb + jo, L)]
            lb = tbl_v[pl.ds(1 * TBL_WORDS + mb + jo, L)]
            sw = tbl_v[pl.ds(2 * TBL_WORDS + mb + jo, L)]
            sb = tbl_v[pl.ds(3 * TBL_WORDS + mb + jo, L)]
            ol1_v[sl] = lb + lw * mu1_v[sl]
            os1_v[sl] = jnp.maximum(sb + sw * sig1_v[sl], REG)
            lw = tbl_v[pl.ds(4 * TBL_WORDS + mb + jo, L)]
            lb = tbl_v[pl.ds(5 * TBL_WORDS + mb + jo, L)]
            sw = tbl_v[pl.ds(6 * TBL_WORDS + mb + jo, L)]
            sb = tbl_v[pl.ds(7 * TBL_WORDS + mb + jo, L)]
            ol2_v[sl] = lb + lw * mu2_v[sl]
            os2_v[sl] = jnp.maximum(sb + sw * sig2_v[sl], REG)

    def store_sample(b, slot):
        for dst, buf in zip(outs, out_bufs):
            pltpu.async_copy(buf.at[pl.ds(slot * CHUNK, CHUNK)],
                             dst.at[pl.ds(b * G + off, CHUNK)],
                             sem_out[slot])

    load_sample(0, 0)
    load_sample(1, 1)

    def body(g, _):
        b = 2 * g
        for slot in (0, 1):
            bs = b + slot
            wait_in(slot)

            @pl.when(bs >= 2)
            def _():
                wait_out(slot)

            compute(bs, slot)
            store_sample(bs, slot)

            @pl.when(bs + 2 < B)
            def _():
                load_sample(bs + 2, slot)

        return 0

    lax.fori_loop(0, B // 2, body, 0)
    wait_out(0)
    wait_out(1)


@functools.partial(jax.jit, static_argnames=())
def _run(t2m_mu, t2m_sig, tp_mu, tp_sig, month, *tables):
    f32 = jnp.float32
    out_type = tuple(jax.ShapeDtypeStruct((B * G,), f32) for _ in range(4))
    mesh = plsc.VectorSubcoreMesh(core_axis_name="c", subcore_axis_name="s",
                                  num_cores=NC, num_subcores=NS)
    scratch = [
        pltpu.VMEM((B + L,), jnp.int32),
        pltpu.VMEM((NT * TBL_WORDS,), f32),
    ] + [pltpu.VMEM((2 * CHUNK,), f32) for _ in range(8)] + [
        pltpu.SemaphoreType.DMA,
        pltpu.SemaphoreType.DMA,
        pltpu.SemaphoreType.DMA,
        pltpu.SemaphoreType.DMA,
    ]
    fn = pl.kernel(_sc_body, out_type=out_type, mesh=mesh,
                   scratch_types=scratch)
    return fn(t2m_mu, t2m_sig, tp_mu, tp_sig, month, *tables)


def kernel(model_parameters_t2m_mu, model_parameters_t2m_sigma,
           model_parameters_tp_mu, model_parameters_tp_sigma, month,
           t2m_loc_w, t2m_loc_b, t2m_scale_w, t2m_scale_b,
           tp_loc_w, tp_loc_b, tp_scale_w, tp_scale_b):
    flat = lambda x: x.reshape(-1)
    outs = _run(flat(model_parameters_t2m_mu),
                flat(model_parameters_t2m_sigma),
                flat(model_parameters_tp_mu),
                flat(model_parameters_tp_sigma),
                month,
                flat(t2m_loc_w), flat(t2m_loc_b),
                flat(t2m_scale_w), flat(t2m_scale_b),
                flat(tp_loc_w), flat(tp_loc_b),
                flat(tp_scale_w), flat(tp_scale_b))
    return tuple(o.reshape(B, H, W) for o in outs)


# speedup vs baseline: 2.2664x; 2.2664x over previous
import jax
import jax.numpy as jnp
from jax import lax
from jax.experimental import pallas as pl
from jax.experimental.pallas import tpu as pltpu
from jax.experimental.pallas import tpu_sc as plsc

B = 256
H, W = 121, 240
NM = 12
NMP = 16
NT = 8
NC, NS, L = 2, 16, 16
NW = NC * NS
WU = 24
NUNIT = (H * W) // WU
NBG = B // L
ROWS_A = 2 * WU
ROWS_S = 4 * ROWS_A
REG = 1e-9


def _worker(ins, outs, tbl2d, month, month_v, idx_v, tbl_v, in_v, out_v,
            sem_in, sem_out, wid):
    n = (NUNIT - wid + NW - 1) // NW

    pltpu.sync_copy(month, month_v.at[pl.ds(0, B)])
    for t in range(NT):
        for g in range(NBG):
            mv = month_v[pl.ds(g * L, L)]
            idx_v[pl.ds((t * NBG + g) * L, L)] = mv + t * NMP

    def unit_row(k):
        u = wid + k * NW
        return pl.multiple_of(u * WU, 8)

    def load_unit(k, slot):
        r = unit_row(k)
        for a in range(4):
            for bt in range(2):
                pltpu.async_copy(
                    ins[a].at[pl.ds(r, WU), pl.ds(bt * 128, 128)],
                    in_v.at[pl.ds(slot * ROWS_S + a * ROWS_A + bt * WU, WU),
                            :],
                    sem_in[slot])
        pltpu.async_copy(tbl2d.at[pl.ds(r * 128, WU * 128)],
                         tbl_v.at[pl.ds(slot * WU * 128, WU * 128)],
                         sem_in[slot])

    def wait_in(slot):
        for _ in range(8):
            pltpu.make_async_copy(
                ins[0].at[pl.ds(0, WU), pl.ds(0, 128)],
                in_v.at[pl.ds(0, WU), :], sem_in[slot]).wait()
        pltpu.make_async_copy(tbl2d.at[pl.ds(0, WU * 128)],
                              tbl_v.at[pl.ds(0, WU * 128)],
                              sem_in[slot]).wait()

    def store_unit(k, slot):
        r = unit_row(k)
        for a in range(4):
            for bt in range(2):
                pltpu.async_copy(
                    out_v.at[pl.ds(slot * ROWS_S + a * ROWS_A + bt * WU, WU),
                             :],
                    outs[a].at[pl.ds(r, WU), pl.ds(bt * 128, 128)],
                    sem_out[slot])

    def wait_out(slot):
        for _ in range(8):
            pltpu.make_async_copy(
                out_v.at[pl.ds(0, WU), :],
                outs[0].at[pl.ds(0, WU), pl.ds(0, 128)],
                sem_out[slot]).wait()

    def compute(slot):
        def bg_body(g, _):
            bt24 = (g >> 3) * WU
            bj16 = (g & 7) * L
            iv = [idx_v[pl.ds(t * (NBG * L) + g * L, L)] for t in range(NT)]
            for wl in range(WU):
                base = (slot * WU + wl) * 128
                lw = plsc.load_gather(tbl_v, [iv[0] + base])
                lb = plsc.load_gather(tbl_v, [iv[1] + base])
                sw = plsc.load_gather(tbl_v, [iv[2] + base])
                sb = plsc.load_gather(tbl_v, [iv[3] + base])
                r0 = slot * ROWS_S + bt24 + wl
                mu = in_v[r0 + 0 * ROWS_A, pl.ds(bj16, L)]
                sg = in_v[r0 + 1 * ROWS_A, pl.ds(bj16, L)]
                out_v[r0 + 0 * ROWS_A, pl.ds(bj16, L)] = lb + lw * mu
                out_v[r0 + 1 * ROWS_A, pl.ds(bj16, L)] = jnp.maximum(
                    sb + sw * sg, REG)
                lw = plsc.load_gather(tbl_v, [iv[4] + base])
                lb = plsc.load_gather(tbl_v, [iv[5] + base])
                sw = plsc.load_gather(tbl_v, [iv[6] + base])
                sb = plsc.load_gather(tbl_v, [iv[7] + base])
                mu = in_v[r0 + 2 * ROWS_A, pl.ds(bj16, L)]
                sg = in_v[r0 + 3 * ROWS_A, pl.ds(bj16, L)]
                out_v[r0 + 2 * ROWS_A, pl.ds(bj16, L)] = lb + lw * mu
                out_v[r0 + 3 * ROWS_A, pl.ds(bj16, L)] = jnp.maximum(
                    sb + sw * sg, REG)
            return 0

        lax.fori_loop(0, NBG, bg_body, 0)

    def step(k, slot):
        wait_in(slot)

        @pl.when(k >= 2)
        def _():
            wait_out(slot)

        compute(slot)
        store_unit(k, slot)

        @pl.when(k + 2 < n)
        def _():
            load_unit(k + 2, slot)

    load_unit(0, 0)
    load_unit(1, 1)

    def pair_body(g, _):
        step(2 * g + 0, 0)
        step(2 * g + 1, 1)
        return 0

    lax.fori_loop(0, n // 2, pair_body, 0)

    @pl.when(n % 2 == 1)
    def _():
        step(n - 1, 0)

    wait_out(0)
    wait_out(1)


def _sc_body(t2m_mu, t2m_sig, tp_mu, tp_sig, month, tbl2d,
             o_t2m_loc, o_t2m_scale, o_tp_loc, o_tp_scale,
             month_v, idx_v, tbl_v, in_v, out_v,
             sem_in0, sem_in1, sem_out0, sem_out1):
    wid = lax.axis_index("s") * NC + lax.axis_index("c")
    _worker((t2m_mu, t2m_sig, tp_mu, tp_sig),
            (o_t2m_loc, o_t2m_scale, o_tp_loc, o_tp_scale),
            tbl2d, month, month_v, idx_v, tbl_v, in_v, out_v,
            (sem_in0, sem_in1), (sem_out0, sem_out1), wid)


@jax.jit
def _run(t2m_mu, t2m_sig, tp_mu, tp_sig, month, tbl2d):
    f32 = jnp.float32
    out_type = tuple(
        jax.ShapeDtypeStruct((H * W, B), f32) for _ in range(4))
    mesh = plsc.VectorSubcoreMesh(core_axis_name="c", subcore_axis_name="s",
                                  num_cores=NC, num_subcores=NS)
    scratch = [
        pltpu.VMEM((B + L,), jnp.int32),
        pltpu.VMEM((NT * NBG * L,), jnp.int32),
        pltpu.VMEM((2 * WU * 128,), f32),
        pltpu.VMEM((2 * ROWS_S, 128), f32),
        pltpu.VMEM((2 * ROWS_S, 128), f32),
        pltpu.SemaphoreType.DMA,
        pltpu.SemaphoreType.DMA,
        pltpu.SemaphoreType.DMA,
        pltpu.SemaphoreType.DMA,
    ]
    fn = pl.kernel(_sc_body, out_type=out_type, mesh=mesh,
                   scratch_types=scratch,
                   compiler_params=pltpu.CompilerParams(
                       needs_layout_passes=False))
    return fn(t2m_mu, t2m_sig, tp_mu, tp_sig, month, tbl2d)


def kernel(model_parameters_t2m_mu, model_parameters_t2m_sigma,
           model_parameters_tp_mu, model_parameters_tp_sigma, month,
           t2m_loc_w, t2m_loc_b, t2m_scale_w, t2m_scale_b,
           tp_loc_w, tp_loc_b, tp_scale_w, tp_scale_b):
    tv = lambda x: jnp.transpose(x, (1, 2, 0)).reshape(H * W, B)
    tbls = jnp.stack([t2m_loc_w, t2m_loc_b, t2m_scale_w, t2m_scale_b,
                      tp_loc_w, tp_loc_b, tp_scale_w, tp_scale_b])
    tbl2d = jnp.pad(tbls.transpose(2, 3, 0, 1), ((0, 0), (0, 0), (0, 0),
                                                 (0, NMP - NM)))
    tbl2d = tbl2d.reshape(-1)
    outs = _run(tv(model_parameters_t2m_mu), tv(model_parameters_t2m_sigma),
                tv(model_parameters_tp_mu), tv(model_parameters_tp_sigma),
                month, tbl2d)
    return tuple(
        jnp.transpose(o.reshape(H, W, B), (2, 0, 1)) for o in outs)


# speedup vs baseline: 3.1228x; 1.3778x over previous
import jax
import jax.numpy as jnp
from jax import lax
from jax.experimental import pallas as pl
from jax.experimental.pallas import tpu as pltpu
from jax.experimental.pallas import tpu_sc as plsc

B = 256
H, W = 121, 240
NM = 12
NMP = 16
NT = 8
NC, NS, L = 2, 16, 16
NW = NC * NS
WU = 24
NUNIT = (H * W) // WU
NBG = B // L
ROWS_A = 2 * WU
ROWS_S = 4 * ROWS_A
REG = 1e-9


def _worker(ins, outs, tbl2d, month, month_v, idx_v, tbl_v, in_v, out_v,
            sem_in, sem_out, wid):
    n = (NUNIT - wid + NW - 1) // NW

    pltpu.sync_copy(month, month_v.at[pl.ds(0, B)])
    for t in range(NT):
        for g in range(NBG):
            mv = month_v[pl.ds(g * L, L)]
            idx_v[pl.ds((t * NBG + g) * L, L)] = mv + t * NMP

    def unit_row(k):
        u = wid + k * NW
        return pl.multiple_of(u * WU, 8)

    def load_unit(k, slot):
        r = unit_row(k)
        for a in range(4):
            for bt in range(2):
                pltpu.async_copy(
                    ins[a].at[pl.ds(r, WU), pl.ds(bt * 128, 128)],
                    in_v.at[pl.ds(slot * ROWS_S + a * ROWS_A + bt * WU, WU),
                            :],
                    sem_in[slot])
        pltpu.async_copy(tbl2d.at[pl.ds(r * 128, WU * 128)],
                         tbl_v.at[pl.ds(slot * WU * 128, WU * 128)],
                         sem_in[slot])

    def wait_in(slot):
        for _ in range(8):
            pltpu.make_async_copy(
                ins[0].at[pl.ds(0, WU), pl.ds(0, 128)],
                in_v.at[pl.ds(0, WU), :], sem_in[slot]).wait()
        pltpu.make_async_copy(tbl2d.at[pl.ds(0, WU * 128)],
                              tbl_v.at[pl.ds(0, WU * 128)],
                              sem_in[slot]).wait()

    def store_unit(k, slot):
        r = unit_row(k)
        for a in range(4):
            for bt in range(2):
                pltpu.async_copy(
                    out_v.at[pl.ds(slot * ROWS_S + a * ROWS_A + bt * WU, WU),
                             :],
                    outs[a].at[pl.ds(r, WU), pl.ds(bt * 128, 128)],
                    sem_out[slot])

    def wait_out(slot):
        for _ in range(8):
            pltpu.make_async_copy(
                out_v.at[pl.ds(0, WU), :],
                outs[0].at[pl.ds(0, WU), pl.ds(0, 128)],
                sem_out[slot]).wait()

    def compute(slot):
        def make_bj_body(bt):
            def bj_body(bj, _):
                bj16 = bj * L
                iv = [idx_v[pl.ds(t * (NBG * L) + bt * (8 * L) + bj16, L)]
                      for t in range(NT)]
                for wl in range(WU):
                    base = (slot * WU + wl) * 128
                    r0 = slot * ROWS_S + bt * WU + wl
                    lw = plsc.load_gather(tbl_v, [iv[0] + base])
                    lb = plsc.load_gather(tbl_v, [iv[1] + base])
                    sw = plsc.load_gather(tbl_v, [iv[2] + base])
                    sb = plsc.load_gather(tbl_v, [iv[3] + base])
                    mu = in_v[r0 + 0 * ROWS_A, pl.ds(bj16, L)]
                    sg = in_v[r0 + 1 * ROWS_A, pl.ds(bj16, L)]
                    out_v[r0 + 0 * ROWS_A, pl.ds(bj16, L)] = lb + lw * mu
                    out_v[r0 + 1 * ROWS_A, pl.ds(bj16, L)] = jnp.maximum(
                        sb + sw * sg, REG)
                    lw = plsc.load_gather(tbl_v, [iv[4] + base])
                    lb = plsc.load_gather(tbl_v, [iv[5] + base])
                    sw = plsc.load_gather(tbl_v, [iv[6] + base])
                    sb = plsc.load_gather(tbl_v, [iv[7] + base])
                    mu = in_v[r0 + 2 * ROWS_A, pl.ds(bj16, L)]
                    sg = in_v[r0 + 3 * ROWS_A, pl.ds(bj16, L)]
                    out_v[r0 + 2 * ROWS_A, pl.ds(bj16, L)] = lb + lw * mu
                    out_v[r0 + 3 * ROWS_A, pl.ds(bj16, L)] = jnp.maximum(
                        sb + sw * sg, REG)
                return 0
            return bj_body

        for bt in range(2):
            lax.fori_loop(0, 8, make_bj_body(bt), 0)

    def step(k, slot):
        wait_in(slot)

        @pl.when(k >= 2)
        def _():
            wait_out(slot)

        compute(slot)
        store_unit(k, slot)

        @pl.when(k + 2 < n)
        def _():
            load_unit(k + 2, slot)

    load_unit(0, 0)
    load_unit(1, 1)

    def pair_body(g, _):
        step(2 * g + 0, 0)
        step(2 * g + 1, 1)
        return 0

    lax.fori_loop(0, n // 2, pair_body, 0)

    @pl.when(n % 2 == 1)
    def _():
        step(n - 1, 0)

    wait_out(0)
    wait_out(1)


def _sc_body(t2m_mu, t2m_sig, tp_mu, tp_sig, month, tbl2d,
             o_t2m_loc, o_t2m_scale, o_tp_loc, o_tp_scale,
             month_v, idx_v, tbl_v, in_v, out_v,
             sem_in0, sem_in1, sem_out0, sem_out1):
    wid = lax.axis_index("s") * NC + lax.axis_index("c")
    _worker((t2m_mu, t2m_sig, tp_mu, tp_sig),
            (o_t2m_loc, o_t2m_scale, o_tp_loc, o_tp_scale),
            tbl2d, month, month_v, idx_v, tbl_v, in_v, out_v,
            (sem_in0, sem_in1), (sem_out0, sem_out1), wid)


@jax.jit
def _run(t2m_mu, t2m_sig, tp_mu, tp_sig, month, tbl2d):
    f32 = jnp.float32
    out_type = tuple(
        jax.ShapeDtypeStruct((H * W, B), f32) for _ in range(4))
    mesh = plsc.VectorSubcoreMesh(core_axis_name="c", subcore_axis_name="s",
                                  num_cores=NC, num_subcores=NS)
    scratch = [
        pltpu.VMEM((B + L,), jnp.int32),
        pltpu.VMEM((NT * NBG * L,), jnp.int32),
        pltpu.VMEM((2 * WU * 128,), f32),
        pltpu.VMEM((2 * ROWS_S, 128), f32),
        pltpu.VMEM((2 * ROWS_S, 128), f32),
        pltpu.SemaphoreType.DMA,
        pltpu.SemaphoreType.DMA,
        pltpu.SemaphoreType.DMA,
        pltpu.SemaphoreType.DMA,
    ]
    fn = pl.kernel(_sc_body, out_type=out_type, mesh=mesh,
                   scratch_types=scratch,
                   compiler_params=pltpu.CompilerParams(
                       needs_layout_passes=False))
    return fn(t2m_mu, t2m_sig, tp_mu, tp_sig, month, tbl2d)


def kernel(model_parameters_t2m_mu, model_parameters_t2m_sigma,
           model_parameters_tp_mu, model_parameters_tp_sigma, month,
           t2m_loc_w, t2m_loc_b, t2m_scale_w, t2m_scale_b,
           tp_loc_w, tp_loc_b, tp_scale_w, tp_scale_b):
    tv = lambda x: jnp.transpose(x, (1, 2, 0)).reshape(H * W, B)
    blocks = []
    for t in (t2m_loc_w, t2m_loc_b, t2m_scale_w, t2m_scale_b,
              tp_loc_w, tp_loc_b, tp_scale_w, tp_scale_b):
        blk = jnp.transpose(t, (1, 2, 0)).reshape(H * W, NM)
        blocks.append(jnp.pad(blk, ((0, 0), (0, NMP - NM))))
    tbl2d = jnp.concatenate(blocks, axis=1).reshape(-1)
    outs = _run(tv(model_parameters_t2m_mu), tv(model_parameters_t2m_sigma),
                tv(model_parameters_tp_mu), tv(model_parameters_tp_sigma),
                month, tbl2d)
    return tuple(
        jnp.transpose(o.reshape(H, W, B), (2, 0, 1)) for o in outs)


# speedup vs baseline: 3.3422x; 1.0703x over previous
import jax
import jax.numpy as jnp
from jax import lax
from jax.experimental import pallas as pl
from jax.experimental.pallas import tpu as pltpu
from jax.experimental.pallas import tpu_sc as plsc

B = 256
H, W = 121, 240
NM = 12
NMP = 16
NT = 8
NC, NS, L = 2, 16, 16
NW = NC * NS
WU = 24
NUNIT = (H * W) // WU
NBG = B // L
ROWS_A = 2 * WU
ROWS_S = 4 * ROWS_A
REG = 1e-9


def _worker(ins, outs, tbl2d, month, month_v, idx_v, tbl_v, in_v, out_v,
            sem_in, sem_out, wid):
    n = (NUNIT - wid + NW - 1) // NW

    pltpu.sync_copy(month, month_v.at[pl.ds(0, B)])
    for t in range(NT):
        for g in range(NBG):
            mv = month_v[pl.ds(g * L, L)]
            idx_v[pl.ds((t * NBG + g) * L, L)] = mv + t * NMP

    def unit_row(k):
        u = wid + k * NW
        return pl.multiple_of(u * WU, 8)

    def load_unit(k, slot):
        r = unit_row(k)
        for a in range(4):
            for bt in range(2):
                pltpu.async_copy(
                    ins[a].at[pl.ds(r, WU), pl.ds(bt * 128, 128)],
                    in_v.at[pl.ds(slot * ROWS_S + a * ROWS_A + bt * WU, WU),
                            :],
                    sem_in[slot])
        pltpu.async_copy(tbl2d.at[pl.ds(r * 128, WU * 128)],
                         tbl_v.at[pl.ds(slot * WU * 128, WU * 128)],
                         sem_in[slot])

    def wait_in(slot):
        for _ in range(8):
            pltpu.make_async_copy(
                ins[0].at[pl.ds(0, WU), pl.ds(0, 128)],
                in_v.at[pl.ds(0, WU), :], sem_in[slot]).wait()
        pltpu.make_async_copy(tbl2d.at[pl.ds(0, WU * 128)],
                              tbl_v.at[pl.ds(0, WU * 128)],
                              sem_in[slot]).wait()

    def store_unit(k, slot):
        r = unit_row(k)
        for a in range(4):
            for bt in range(2):
                pltpu.async_copy(
                    out_v.at[pl.ds(slot * ROWS_S + a * ROWS_A + bt * WU, WU),
                             :],
                    outs[a].at[pl.ds(r, WU), pl.ds(bt * 128, 128)],
                    sem_out[slot])

    def wait_out(slot):
        for _ in range(8):
            pltpu.make_async_copy(
                out_v.at[pl.ds(0, WU), :],
                outs[0].at[pl.ds(0, WU), pl.ds(0, 128)],
                sem_out[slot]).wait()

    def compute(slot):
        def make_bj_body(bt):
            def bj_body(bj, _):
                bj16 = bj * L
                iv = [idx_v[pl.ds(t * (NBG * L) + bt * (8 * L) + bj16, L)]
                      for t in range(NT)]
                for wl in range(WU):
                    base = (slot * WU + wl) * 128
                    r0 = slot * ROWS_S + bt * WU + wl
                    lw = plsc.load_gather(tbl_v, [iv[0] + base])
                    lb = plsc.load_gather(tbl_v, [iv[1] + base])
                    sw = plsc.load_gather(tbl_v, [iv[2] + base])
                    sb = plsc.load_gather(tbl_v, [iv[3] + base])
                    mu = in_v[r0 + 0 * ROWS_A, pl.ds(bj16, L)]
                    sg = in_v[r0 + 1 * ROWS_A, pl.ds(bj16, L)]
                    out_v[r0 + 0 * ROWS_A, pl.ds(bj16, L)] = lb + lw * mu
                    out_v[r0 + 1 * ROWS_A, pl.ds(bj16, L)] = jnp.maximum(
                        sb + sw * sg, REG)
                    lw = plsc.load_gather(tbl_v, [iv[4] + base])
                    lb = plsc.load_gather(tbl_v, [iv[5] + base])
                    sw = plsc.load_gather(tbl_v, [iv[6] + base])
                    sb = plsc.load_gather(tbl_v, [iv[7] + base])
                    mu = in_v[r0 + 2 * ROWS_A, pl.ds(bj16, L)]
                    sg = in_v[r0 + 3 * ROWS_A, pl.ds(bj16, L)]
                    out_v[r0 + 2 * ROWS_A, pl.ds(bj16, L)] = lb + lw * mu
                    out_v[r0 + 3 * ROWS_A, pl.ds(bj16, L)] = jnp.maximum(
                        sb + sw * sg, REG)
                return 0
            return bj_body

        for bt in range(2):
            lax.fori_loop(0, 8, make_bj_body(bt), 0)

    def step(k, slot):
        wait_in(slot)

        @pl.when(k >= 2)
        def _():
            wait_out(slot)

        compute(slot)
        store_unit(k, slot)

        @pl.when(k + 2 < n)
        def _():
            load_unit(k + 2, slot)

    load_unit(0, 0)
    load_unit(1, 1)

    def pair_body(g, _):
        step(2 * g + 0, 0)
        step(2 * g + 1, 1)
        return 0

    lax.fori_loop(0, n // 2, pair_body, 0)

    @pl.when(n % 2 == 1)
    def _():
        step(n - 1, 0)

    wait_out(0)
    wait_out(1)


def _sc_body(t2m_mu, t2m_sig, tp_mu, tp_sig, month, tbl2d,
             o_t2m_loc, o_t2m_scale, o_tp_loc, o_tp_scale,
             month_v, idx_v, tbl_v, in_v, out_v,
             sem_in0, sem_in1, sem_out0, sem_out1):
    wid = lax.axis_index("s") * NC + lax.axis_index("c")
    _worker((t2m_mu, t2m_sig, tp_mu, tp_sig),
            (o_t2m_loc, o_t2m_scale, o_tp_loc, o_tp_scale),
            tbl2d, month, month_v, idx_v, tbl_v, in_v, out_v,
            (sem_in0, sem_in1), (sem_out0, sem_out1), wid)


@jax.jit
def _run(t2m_mu, t2m_sig, tp_mu, tp_sig, month, tbl2d):
    f32 = jnp.float32
    out_type = tuple(
        jax.ShapeDtypeStruct((H * W, B), f32) for _ in range(4))
    mesh = plsc.VectorSubcoreMesh(core_axis_name="c", subcore_axis_name="s",
                                  num_cores=NC, num_subcores=NS)
    scratch = [
        pltpu.VMEM((B + L,), jnp.int32),
        pltpu.VMEM((NT * NBG * L,), jnp.int32),
        pltpu.VMEM((2 * WU * 128,), f32),
        pltpu.VMEM((2 * ROWS_S, 128), f32),
        pltpu.VMEM((2 * ROWS_S, 128), f32),
        pltpu.SemaphoreType.DMA,
        pltpu.SemaphoreType.DMA,
        pltpu.SemaphoreType.DMA,
        pltpu.SemaphoreType.DMA,
    ]
    fn = pl.kernel(_sc_body, out_type=out_type, mesh=mesh,
                   scratch_types=scratch,
                   compiler_params=pltpu.CompilerParams(
                       needs_layout_passes=False))
    return fn(t2m_mu, t2m_sig, tp_mu, tp_sig, month, tbl2d)


def kernel(model_parameters_t2m_mu, model_parameters_t2m_sigma,
           model_parameters_tp_mu, model_parameters_tp_sigma, month,
           t2m_loc_w, t2m_loc_b, t2m_scale_w, t2m_scale_b,
           tp_loc_w, tp_loc_b, tp_scale_w, tp_scale_b):
    tv = lambda x: jnp.transpose(x, (1, 2, 0)).reshape(H * W, B)
    cat = jnp.concatenate([t2m_loc_w, t2m_loc_b, t2m_scale_w, t2m_scale_b,
                           tp_loc_w, tp_loc_b, tp_scale_w, tp_scale_b])
    m1 = cat.reshape(NT * NM, H * W)
    k_idx = jnp.arange(NT * NM)
    c_idx = (k_idx // NM) * NMP + (k_idx % NM)
    p = jax.nn.one_hot(c_idx, NT * NMP, dtype=jnp.float32)
    tbl2d = jax.lax.dot_general(
        m1, p, (((0,), (0,)), ((), ())),
        precision=jax.lax.Precision.HIGHEST,
        preferred_element_type=jnp.float32)
    tbl2d = tbl2d.reshape(-1)
    outs = _run(tv(model_parameters_t2m_mu), tv(model_parameters_t2m_sigma),
                tv(model_parameters_tp_mu), tv(model_parameters_tp_sigma),
                month, tbl2d)
    return tuple(
        jnp.transpose(o.reshape(H, W, B), (2, 0, 1)) for o in outs)
